# baseline (speedup 1.0000x reference)
import jax
import jax.numpy as jnp
from jax import lax
from jax.experimental import pallas as pl
from jax.experimental.pallas import tpu as pltpu
from jax.experimental.pallas import tpu_sc as plsc

VOCAB = 50000
FULL_VOCAB = 100000
ROWS = 512
NW = 32
RPW = ROWS // NW
LANES = 16
NVREG = VOCAB // LANES


def _sc_body(logits_hbm, out_hbm, row_buf, res_ref):
    c = lax.axis_index("c")
    s = lax.axis_index("s")
    wid = c * 16 + s
    base = wid * RPW
    lane = lax.iota(jnp.int32, LANES)
    big = jnp.full((LANES,), 2**30, jnp.int32)

    res = jnp.zeros((LANES,), jnp.int32)
    for r in range(RPW):
        row = base + r
        pltpu.sync_copy(logits_hbm.at[pl.ds(row * FULL_VOCAB, VOCAB)], row_buf)

        def body(j, carry):
            m, mi = carry
            v = row_buf[pl.ds(j * LANES, LANES)]
            col = j * LANES + lane
            upd = v > m
            return jnp.where(upd, v, m), jnp.where(upd, col, mi)

        m0 = jnp.full((LANES,), -jnp.inf, jnp.float32)
        i0 = jnp.zeros((LANES,), jnp.int32)
        m, mi = lax.fori_loop(0, NVREG, body, (m0, i0))

        mx = jnp.max(m)
        best = jnp.min(jnp.where(m == mx, mi, big))
        res = jnp.where(lane == r, best, res)

    res_ref[...] = res
    pltpu.sync_copy(res_ref, out_hbm.at[pl.ds(base, RPW)])


def kernel(logits):
    x = logits.reshape(ROWS * logits.shape[-1])
    mesh = plsc.VectorSubcoreMesh(core_axis_name="c", subcore_axis_name="s")
    out = pl.kernel(
        _sc_body,
        out_type=jax.ShapeDtypeStruct((ROWS,), jnp.int32),
        mesh=mesh,
        compiler_params=pltpu.CompilerParams(needs_layout_passes=False),
        scratch_types=[
            pltpu.VMEM((VOCAB,), jnp.float32),
            pltpu.VMEM((LANES,), jnp.int32),
        ],
    )(x)
    return out.reshape(logits.shape[0], logits.shape[1])


# speedup vs baseline: 1.4195x; 1.4195x over previous
import jax
import jax.numpy as jnp
from jax import lax
from jax.experimental import pallas as pl
from jax.experimental.pallas import tpu as pltpu
from jax.experimental.pallas import tpu_sc as plsc

VOCAB = 50000
FULL_VOCAB = 100000
ROWS = 512
NW = 32
RPW = ROWS // NW
LANES = 16
NVREG = VOCAB // LANES
NSUB = 25
SUBC = VOCAB // NSUB
SUBV = SUBC // LANES
UNROLL = 25
NACC = 5
AIT = SUBV // UNROLL
BUN = 5
BIT = SUBV // BUN
BIG = 2**30


_GATHER_DNUMS = lax.GatherDimensionNumbers(
    offset_dims=(), collapsed_slice_dims=(0,), start_index_map=(0,))


def _xlane(x, idx):
    return lax.gather(x, idx[:, None], _GATHER_DNUMS, (1,),
                      mode=lax.GatherScatterMode.PROMISE_IN_BOUNDS)


def _lane_bcast(x, perms, combine):
    for p in perms:
        x = combine(x, _xlane(x, p))
    return x


def _sc_body(logits_hbm, out_hbm, buf0, buf1, submax_ref, res_ref,
             sem0, sem1):
    c = lax.axis_index("c")
    s = lax.axis_index("s")
    wid = c * 16 + s
    base = wid * RPW
    lane = lax.iota(jnp.int32, LANES)
    perms = [lane ^ st for st in (8, 4, 2, 1)]
    neg = jnp.full((LANES,), -jnp.inf, jnp.float32)
    bigv = jnp.full((LANES,), BIG, jnp.int32)

    bufs = (buf0, buf1)
    sems = (sem0, sem1)

    def copy_row(r):
        return pltpu.async_copy(
            logits_hbm.at[pl.ds((base + r) * FULL_VOCAB, VOCAB)],
            bufs[r % 2], sems[r % 2])

    pending = copy_row(0)
    res = jnp.zeros((LANES,), jnp.int32)
    for r in range(RPW):
        nxt = copy_row(r + 1) if r + 1 < RPW else None
        pending.wait()
        pending = nxt
        buf = bufs[r % 2]

        def sub_body(sidx, gm):
            boff = sidx * SUBC

            def inner(j, accs):
                off = boff + j * (UNROLL * LANES)
                accs = list(accs)
                for u in range(UNROLL):
                    v = buf[pl.ds(off + u * LANES, LANES)]
                    accs[u % NACC] = jnp.maximum(accs[u % NACC], v)
                return tuple(accs)

            accs = lax.fori_loop(0, AIT, inner, (neg,) * NACC)
            mm = accs[0]
            for a in accs[1:]:
                mm = jnp.maximum(mm, a)
            submax_ref[pl.ds(sidx * LANES, LANES)] = mm
            return jnp.maximum(gm, mm)

        gm = lax.fori_loop(0, NSUB, sub_body, neg)
        mvec = _lane_bcast(gm, perms, jnp.maximum)

        def find_body(sidx, bsub):
            mm = submax_ref[pl.ds(sidx * LANES, LANES)]
            cnt = plsc.all_reduce_population_count(mm == mvec)
            hit = (cnt > 0) & (bsub < 0)
            return jnp.where(hit, sidx, bsub)

        bsub = lax.fori_loop(0, NSUB, find_body,
                             jnp.full((LANES,), -1, jnp.int32))
        cb = bsub[0] * SUBC

        def b_body(j, mn):
            off = cb + j * (BUN * LANES)
            for u in range(BUN):
                o = off + u * LANES
                v = buf[pl.ds(o, LANES)]
                mn = jnp.minimum(mn, jnp.where(v == mvec, o + lane, mn))
            return mn

        mn = lax.fori_loop(0, BIT, b_body, bigv)
        mi = _lane_bcast(mn, perms, jnp.minimum)
        res = jnp.where(lane == r, mi, res)

    res_ref[...] = res
    pltpu.sync_copy(res_ref, out_hbm.at[pl.ds(base, RPW)])


def kernel(logits):
    x = logits.reshape(ROWS * logits.shape[-1])
    mesh = plsc.VectorSubcoreMesh(core_axis_name="c", subcore_axis_name="s")
    out = pl.kernel(
        _sc_body,
        out_type=jax.ShapeDtypeStruct((ROWS,), jnp.int32),
        mesh=mesh,
        compiler_params=pltpu.CompilerParams(needs_layout_passes=False),
        scratch_types=[
            pltpu.VMEM((VOCAB,), jnp.float32),
            pltpu.VMEM((VOCAB,), jnp.float32),
            pltpu.VMEM((NSUB * LANES,), jnp.float32),
            pltpu.VMEM((LANES,), jnp.int32),
            pltpu.SemaphoreType.DMA,
            pltpu.SemaphoreType.DMA,
        ],
    )(x)
    return out.reshape(logits.shape[0], logits.shape[1])


# speedup vs baseline: 2.5574x; 1.8017x over previous
import jax
import jax.numpy as jnp
from jax import lax
from jax.experimental import pallas as pl
from jax.experimental.pallas import tpu as pltpu
from jax.experimental.pallas import tpu_sc as plsc

VOCAB = 50000
NI = 128
NR = 4
NW = 32
GPW = NI // NW
LANES = 16
SUBC = 896
NSUBC = 4 * 14
CHUNK = SUBC * 14
NCH = 4
FULLV = 45
NACC = 8
BIG = 2**30


_GATHER_DNUMS = lax.GatherDimensionNumbers(
    offset_dims=(), collapsed_slice_dims=(0,), start_index_map=(0,))


def _xlane(x, idx):
    return lax.gather(x, idx[:, None], _GATHER_DNUMS, (1,),
                      mode=lax.GatherScatterMode.PROMISE_IN_BOUNDS)


def _lane_bcast(x, perms, combine):
    for p in perms:
        x = combine(x, _xlane(x, p))
    return x


def _fold_max(buf, r, off, nv):
    accs = [buf[r, pl.ds(off + u * LANES, LANES)] for u in range(NACC)]
    for u in range(NACC, nv):
        accs[u % NACC] = jnp.maximum(accs[u % NACC],
                                     buf[r, pl.ds(off + u * LANES, LANES)])
    mm = accs[0]
    for a in accs[1:]:
        mm = jnp.maximum(mm, a)
    return mm


def _sc_body(logits_hbm, out_hbm, bufa, bufb, submax_ref, res_ref,
             sema, semb, semp, pbuf):
    c = lax.axis_index("c")
    s = lax.axis_index("s")
    wid = c * 16 + s
    ibase = wid * GPW
    lane = lax.iota(jnp.int32, LANES)
    perms = [lane ^ st for st in (8, 4, 2, 1)]
    bufs = (bufa, bufb)
    sems = (sema, semb)

    def chunk_copy(g, k):
        return pltpu.async_copy(
            logits_hbm.at[ibase + g, :, pl.ds(k * CHUNK, CHUNK)],
            bufs[k % 2], sems[k % 2])

    def passa_chunk(k):
        buf = bufs[k % 2]
        nsub = 14 if k < NCH - 1 else 13

        def row_body(r, _):
            def sub_body(si, _):
                mm = _fold_max(buf, r, si * SUBC, SUBC // LANES)
                submax_ref[pl.ds(((r * NSUBC) + (k * 14 + si)) * LANES,
                                 LANES)] = mm
                return 0

            lax.fori_loop(0, nsub, sub_body, 0)
            if k == NCH - 1:
                mm = _fold_max(buf, r, 13 * SUBC, FULLV)
                submax_ref[pl.ds(((r * NSUBC) + 55) * LANES, LANES)] = mm
            return 0

        lax.fori_loop(0, NR, row_body, 0)

    res = jnp.zeros((LANES,), jnp.int32)
    nxt = [None, None]
    nxt[0] = chunk_copy(0, 0)
    nxt[1] = chunk_copy(0, 1)
    for g in range(GPW):
        cur, nxt = nxt, [None, None]
        for k in range(NCH):
            cur[k % 2].wait()
            passa_chunk(k)
            if k + 2 < NCH:
                cur[k % 2] = chunk_copy(g, k + 2)
            elif k + 2 == NCH and g + 1 < GPW:
                nxt[k % 2] = chunk_copy(g + 1, 0)

        mvecs, pcopies, bcols = [], [], []
        for r in range(NR):
            gm = submax_ref[pl.ds(r * NSUBC * LANES, LANES)]

            def gm_body(si, acc):
                return jnp.maximum(
                    acc, submax_ref[pl.ds((r * NSUBC + si) * LANES, LANES)])

            gm = lax.fori_loop(1, NSUBC, gm_body, gm)
            mvec = _lane_bcast(gm, perms, jnp.maximum)
            mvecs.append(mvec)

            def find_body(j, mn):
                for u in range(NACC):
                    si = j * NACC + u
                    mm = submax_ref[pl.ds((r * NSUBC + si) * LANES, LANES)]
                    mn = jnp.minimum(mn, jnp.where(mm == mvec, si, BIG))
                return mn

            mns = lax.fori_loop(0, NSUBC // NACC, find_body,
                                jnp.full((LANES,), BIG, jnp.int32))
            bs = _lane_bcast(mns, perms, jnp.minimum)
            bcol = pl.multiple_of(bs[0] * SUBC, 128)
            bcols.append(bcol)
            pcopies.append(pltpu.async_copy(
                logits_hbm.at[ibase + g, :, pl.ds(bcol, SUBC)],
                pbuf.at[r], semp))

        if g + 1 < GPW:
            nxt[1] = chunk_copy(g + 1, 1)

        for r in range(NR):
            pcopies[r].wait()
            mvec, bcol = mvecs[r], bcols[r]

            def b_body(j, mn):
                for u in range(NACC):
                    o = (j * NACC + u) * LANES
                    v = pbuf[r, r, pl.ds(o, LANES)]
                    col = bcol + o + lane
                    hit = (v == mvec) & (col < VOCAB)
                    mn = jnp.minimum(mn, jnp.where(hit, col, BIG))
                return mn

            mns = lax.fori_loop(0, SUBC // LANES // NACC, b_body,
                               jnp.full((LANES,), BIG, jnp.int32))
            mi = _lane_bcast(mns, perms, jnp.minimum)
            res = jnp.where(lane == g * NR + r, mi, res)

    res_ref[...] = res
    pltpu.sync_copy(res_ref, out_hbm.at[pl.ds(wid * GPW * NR, GPW * NR)])


def kernel(logits):
    mesh = plsc.VectorSubcoreMesh(core_axis_name="c", subcore_axis_name="s")
    out = pl.kernel(
        _sc_body,
        out_type=jax.ShapeDtypeStruct((NI * NR,), jnp.int32),
        mesh=mesh,
        compiler_params=pltpu.CompilerParams(needs_layout_passes=False),
        scratch_types=[
            pltpu.VMEM((NR, CHUNK), jnp.float32),
            pltpu.VMEM((NR, CHUNK), jnp.float32),
            pltpu.VMEM((NR * NSUBC * LANES,), jnp.float32),
            pltpu.VMEM((LANES,), jnp.int32),
            pltpu.SemaphoreType.DMA,
            pltpu.SemaphoreType.DMA,
            pltpu.SemaphoreType.DMA,
            pltpu.VMEM((NR, NR, SUBC), jnp.float32),
        ],
    )(logits)
    return out.reshape(logits.shape[0], logits.shape[1])


# speedup vs baseline: 11.9699x; 4.6805x over previous
import jax
import jax.numpy as jnp
from jax import lax
from jax.experimental import pallas as pl
from jax.experimental.pallas import tpu as pltpu
from jax.experimental.pallas import tpu_sc as plsc

VOCAB = 50000
NQ = 4
NB = 128
LANES = 16
COLV = NB // LANES
NSH = 8
SHARD = 6272
CH = 448
NCHK = SHARD // CH
LASTN = VOCAB - 7 * SHARD - 13 * CH


def _sc_body(x_hbm, out_hbm, bufa, bufb, stage_m, stage_i, merge_m, merge_i,
             shm, shi, sema, semb):
    c = lax.axis_index("c")
    s = lax.axis_index("s")
    q = c * 2 + s // NSH
    w = s % NSH
    v0 = w * SHARD
    bufs = (bufa, bufb)
    sems = (sema, semb)

    def chunk_copy(k):
        return pltpu.async_copy(
            x_hbm.at[q, pl.ds(v0 + k * CH, CH), :], bufs[k % 2], sems[k % 2])

    neg = jnp.full((LANES,), -jnp.inf, jnp.float32)
    zero = jnp.zeros((LANES,), jnp.int32)
    ms = [neg] * COLV
    mis = [zero] * COLV

    pending = [chunk_copy(0), chunk_copy(1)]
    for k in range(NCHK):
        pending[k % 2].wait()
        buf = bufs[k % 2]
        base = v0 + k * CH

        def row_body(vr, carry):
            ms, mis = list(carry[0]), list(carry[1])
            vsp = jnp.full((LANES,), base + vr, jnp.int32)
            for u in range(COLV):
                v = buf[vr, pl.ds(u * LANES, LANES)]
                upd = v > ms[u]
                ms[u] = jnp.where(upd, v, ms[u])
                mis[u] = jnp.where(upd, vsp, mis[u])
            return tuple(ms), tuple(mis)

        if k < NCHK - 1:
            nrows = CH
        else:
            nrows = jnp.where(w == NSH - 1, LASTN, CH)
        ms, mis = lax.fori_loop(0, nrows, row_body, (tuple(ms), tuple(mis)))
        ms, mis = list(ms), list(mis)
        if k + 2 < NCHK:
            pending[k % 2] = chunk_copy(k + 2)

    for u in range(COLV):
        stage_m[pl.ds(u * LANES, LANES)] = ms[u]
        stage_i[pl.ds(u * LANES, LANES)] = mis[u]
    pltpu.sync_copy(stage_m, shm.at[s])
    pltpu.sync_copy(stage_i, shi.at[s])
    plsc.subcore_barrier()

    @pl.when(w == 0)
    def _():
        msf = list(ms)
        misf = list(mis)
        for sh in range(1, NSH):
            pltpu.sync_copy(shm.at[s + sh], merge_m)
            pltpu.sync_copy(shi.at[s + sh], merge_i)
            for u in range(COLV):
                mv = merge_m[pl.ds(u * LANES, LANES)]
                iv = merge_i[pl.ds(u * LANES, LANES)]
                take = mv > msf[u]
                msf[u] = jnp.where(take, mv, msf[u])
                misf[u] = jnp.where(take, iv, misf[u])
        for u in range(COLV):
            stage_i[pl.ds(u * LANES, LANES)] = misf[u]
        pltpu.sync_copy(stage_i, out_hbm.at[pl.ds(q * NB, NB)])


def kernel(logits):
    xt = jnp.transpose(logits, (1, 2, 0))
    mesh = plsc.VectorSubcoreMesh(core_axis_name="c", subcore_axis_name="s")
    out = pl.kernel(
        _sc_body,
        out_type=jax.ShapeDtypeStruct((NQ * NB,), jnp.int32),
        mesh=mesh,
        compiler_params=pltpu.CompilerParams(needs_layout_passes=False),
        scratch_types=[
            pltpu.VMEM((CH, NB), jnp.float32),
            pltpu.VMEM((CH, NB), jnp.float32),
            pltpu.VMEM((NB,), jnp.float32),
            pltpu.VMEM((NB,), jnp.int32),
            pltpu.VMEM((NB,), jnp.float32),
            pltpu.VMEM((NB,), jnp.int32),
            pltpu.VMEM_SHARED((16, NB), jnp.float32),
            pltpu.VMEM_SHARED((16, NB), jnp.int32),
            pltpu.SemaphoreType.DMA,
            pltpu.SemaphoreType.DMA,
        ],
    )(xt)
    return out.reshape(NQ, NB).T
